# baseline (speedup 1.0000x reference)
import functools

import jax
import jax.numpy as jnp
from jax import lax
from jax.experimental import pallas as pl
from jax.experimental.pallas import tpu as pltpu
from jax.experimental.pallas import tpu_sc as plsc

N = 10000
E = 320000
D = 128
DH = D // 2
NC = 2
NS = 16
EPT = E // NS
CHUNK = 80
NCHUNK = 250
EPTP = NCHUNK * CHUNK
PAD = EPTP - EPT
NBUF = 2
NP = 10240
RPT = NP // NS


def _sc_aggregate(x0, x1, eidx4):
    mesh = plsc.VectorSubcoreMesh(core_axis_name="c", subcore_axis_name="s")

    @functools.partial(
        pl.kernel,
        out_type=[
            jax.ShapeDtypeStruct((NC, NP, DH), jnp.float32),
            jax.ShapeDtypeStruct((NC, NP, 16), jnp.float32),
        ],
        mesh=mesh,
        compiler_params=pltpu.CompilerParams(use_tc_tiling_on_sc=False),
        scratch_types=[
            pltpu.VMEM((NCHUNK + NBUF, CHUNK), jnp.int32),
            pltpu.VMEM((NCHUNK, CHUNK), jnp.int32),
            [pltpu.VMEM((CHUNK, DH), jnp.float32)] * NBUF,
            pltpu.VMEM((CHUNK, 16), jnp.float32),
            pltpu.VMEM((CHUNK, 16), jnp.float32),
            pltpu.VMEM_SHARED((NP, DH), jnp.float32),
            pltpu.VMEM_SHARED((NP, 16), jnp.float32),
            [pltpu.SemaphoreType.DMA] * NBUF,
            [pltpu.SemaphoreType.DMA] * NBUF,
            pltpu.SemaphoreType.DMA,
        ],
    )
    def body(x0_hbm, x1_hbm, eidx_hbm, agg_out, deg_out,
             srcb, dstb, bufs, ones, zer16, agg_sh, deg_sh,
             gsems, ssems, dsem):
        c = lax.axis_index("c")
        s = lax.axis_index("s")

        zv = jnp.zeros((16,), jnp.float32)
        zvi = jnp.zeros((16,), jnp.int32)
        ov = jnp.ones((16,), jnp.float32)

        def zfill(i, carry):
            bufs[0][i // 4, pl.ds((i % 4) * 16, 16)] = zv
            return carry
        lax.fori_loop(0, CHUNK * 4, zfill, 0)

        for r in range(NCHUNK, NCHUNK + NBUF):
            for j in range(CHUNK // 16):
                srcb[r, pl.ds(j * 16, 16)] = zvi

        def z16fill(i, carry):
            zer16[i] = zv
            return carry
        lax.fori_loop(0, CHUNK, z16fill, 0)

        def ofill(i, carry):
            ones[i] = ov
            return carry
        lax.fori_loop(0, CHUNK, ofill, 0)

        init_handles = [
            pltpu.async_copy(eidx_hbm.at[0, s], srcb.at[pl.ds(0, NCHUNK)],
                             ssems[0]),
            pltpu.async_copy(eidx_hbm.at[1, s], dstb, ssems[1]),
        ]
        for j in range(RPT // CHUNK):
            base = s * RPT + j * CHUNK
            init_handles.append(pltpu.async_copy(
                bufs[0], agg_sh.at[pl.ds(base, CHUNK)], gsems[0]))
            init_handles.append(pltpu.async_copy(
                zer16, deg_sh.at[pl.ds(base, CHUNK)], gsems[1]))
        for h in init_handles:
            h.wait()

        plsc.subcore_barrier()

        def run_loop(x_hbm, deg_par):
            rows0, rows1 = bufs[0], bufs[1]
            sem0, sem1 = gsems[0], gsems[1]
            pltpu.async_copy(x_hbm.at[srcb.at[0]], rows0, sem0)

            def pair(p, carry):
                t0 = 2 * p
                g1 = pltpu.async_copy(x_hbm.at[srcb.at[t0 + 1]], rows1, sem1)
                pltpu.make_async_copy(x_hbm.at[srcb.at[t0]], rows0,
                                      sem0).wait()
                pltpu.sync_copy(rows0, agg_sh.at[dstb.at[t0]], add=True)
                if deg_par == 0:
                    pltpu.sync_copy(ones, deg_sh.at[dstb.at[t0]], add=True)
                pltpu.async_copy(x_hbm.at[srcb.at[t0 + 2]], rows0, sem0)
                g1.wait()
                pltpu.sync_copy(rows1, agg_sh.at[dstb.at[t0 + 1]], add=True)
                if deg_par == 1:
                    pltpu.sync_copy(ones, deg_sh.at[dstb.at[t0 + 1]],
                                    add=True)
                return carry
            lax.fori_loop(0, NCHUNK // 2, pair, 0)
            pltpu.make_async_copy(x_hbm.at[srcb.at[NCHUNK]], rows0,
                                  sem0).wait()

        @pl.when(c == 0)
        def _():
            run_loop(x0_hbm, 0)

        @pl.when(c == 1)
        def _():
            run_loop(x1_hbm, 1)

        plsc.subcore_barrier()

        wb0 = pltpu.async_copy(agg_sh.at[pl.ds(s * RPT, RPT)],
                               agg_out.at[c, pl.ds(s * RPT, RPT)], gsems[0])
        wb1 = pltpu.async_copy(deg_sh.at[pl.ds(s * RPT, RPT)],
                               deg_out.at[c, pl.ds(s * RPT, RPT)], gsems[1])
        wb0.wait()
        wb1.wait()

    return body(x0, x1, eidx4)


BN = 10000


def _tc_split(x):
    def body(x_ref, o0_ref, o1_ref):
        o0_ref[...] = x_ref[:, 0:DH]
        o1_ref[...] = x_ref[:, DH:D]

    return pl.pallas_call(
        body,
        grid=(N // BN,),
        in_specs=[pl.BlockSpec((BN, D), lambda i: (i, 0))],
        out_specs=[pl.BlockSpec((BN, DH), lambda i: (i, 0)),
                   pl.BlockSpec((BN, DH), lambda i: (i, 0))],
        out_shape=[jax.ShapeDtypeStruct((N, DH), jnp.float32),
                   jax.ShapeDtypeStruct((N, DH), jnp.float32)],
    )(x)


def _tc_combine(aggsum, degbuf, x, wl_t, wr_t, b_row):
    def body(agg_ref, deg_ref, x_ref, wl_ref, wr_ref, b_ref, o_ref):
        d = deg_ref[0, :, 0:1] + deg_ref[1, :, 0:1]
        inv = 1.0 / jnp.maximum(d, 1.0)
        a0 = agg_ref[0] * inv
        a1 = agg_ref[1] * inv
        out = jnp.dot(a0, wl_ref[0:DH, :], preferred_element_type=jnp.float32)
        out = out + jnp.dot(a1, wl_ref[DH:D, :],
                            preferred_element_type=jnp.float32)
        out = out + jnp.dot(x_ref[...], wr_ref[...],
                            preferred_element_type=jnp.float32)
        out = out + b_ref[...]
        o_ref[...] = jnp.maximum(out, 0.0)

    return pl.pallas_call(
        body,
        grid=(N // BN,),
        in_specs=[
            pl.BlockSpec((NC, BN, DH), lambda i: (0, i, 0)),
            pl.BlockSpec((NC, BN, 16), lambda i: (0, i, 0)),
            pl.BlockSpec((BN, D), lambda i: (i, 0)),
            pl.BlockSpec((D, D), lambda i: (0, 0)),
            pl.BlockSpec((D, D), lambda i: (0, 0)),
            pl.BlockSpec((1, D), lambda i: (0, 0)),
        ],
        out_specs=pl.BlockSpec((BN, D), lambda i: (i, 0)),
        out_shape=jax.ShapeDtypeStruct((N, D), jnp.float32),
    )(aggsum, degbuf, x, wl_t, wr_t, b_row)


@jax.jit
def kernel(x, edge_index, W_l, b_l, W_r):
    x0, x1 = _tc_split(x)
    eidx4 = edge_index.reshape(2, NS, NCHUNK, CHUNK)
    aggsum, degbuf = _sc_aggregate(x0, x1, eidx4)
    return _tc_combine(aggsum, degbuf, x, W_l.T, W_r.T, b_l.reshape(1, D))


# speedup vs baseline: 1.0039x; 1.0039x over previous
import functools

import jax
import jax.numpy as jnp
from jax import lax
from jax.experimental import pallas as pl
from jax.experimental.pallas import tpu as pltpu
from jax.experimental.pallas import tpu_sc as plsc

N = 10000
E = 320000
D = 128
DH = D // 2
NC = 2
NS = 16
EPT = E // NS
CHUNK = 80
NCHUNK = 250
EPTP = NCHUNK * CHUNK
PAD = EPTP - EPT
NBUF = 2
NP = 10240
RPT = NP // NS


def _sc_aggregate(x0, x1, eidx4):
    mesh = plsc.VectorSubcoreMesh(core_axis_name="c", subcore_axis_name="s")

    @functools.partial(
        pl.kernel,
        out_type=[
            jax.ShapeDtypeStruct((NC, NP, DH), jnp.float32),
            jax.ShapeDtypeStruct((NC, NP, 16), jnp.float32),
        ],
        mesh=mesh,
        compiler_params=pltpu.CompilerParams(use_tc_tiling_on_sc=False),
        scratch_types=[
            pltpu.VMEM((NCHUNK + NBUF, CHUNK), jnp.int32),
            pltpu.VMEM((NCHUNK, CHUNK), jnp.int32),
            [pltpu.VMEM((CHUNK, DH), jnp.float32)] * NBUF,
            pltpu.VMEM((CHUNK, 16), jnp.float32),
            pltpu.VMEM((CHUNK, 16), jnp.float32),
            pltpu.VMEM_SHARED((NP, DH), jnp.float32),
            pltpu.VMEM_SHARED((NP, 16), jnp.float32),
            [pltpu.SemaphoreType.DMA] * NBUF,
            [pltpu.SemaphoreType.DMA] * NBUF,
            pltpu.SemaphoreType.DMA,
        ],
    )
    def body(x0_hbm, x1_hbm, eidx_hbm, agg_out, deg_out,
             srcb, dstb, bufs, ones, zer16, agg_sh, deg_sh,
             gsems, ssems, dsem):
        c = lax.axis_index("c")
        s = lax.axis_index("s")

        zv = jnp.zeros((16,), jnp.float32)
        zvi = jnp.zeros((16,), jnp.int32)
        ov = jnp.ones((16,), jnp.float32)

        def zfill(i, carry):
            bufs[0][i // 4, pl.ds((i % 4) * 16, 16)] = zv
            return carry
        lax.fori_loop(0, CHUNK * 4, zfill, 0)

        for r in range(NCHUNK, NCHUNK + NBUF):
            for j in range(CHUNK // 16):
                srcb[r, pl.ds(j * 16, 16)] = zvi

        def z16fill(i, carry):
            zer16[i] = zv
            return carry
        lax.fori_loop(0, CHUNK, z16fill, 0)

        def ofill(i, carry):
            ones[i] = ov
            return carry
        lax.fori_loop(0, CHUNK, ofill, 0)

        init_handles = [
            pltpu.async_copy(eidx_hbm.at[0, s], srcb.at[pl.ds(0, NCHUNK)],
                             ssems[0]),
            pltpu.async_copy(eidx_hbm.at[1, s], dstb, ssems[1]),
        ]
        for j in range(RPT // CHUNK):
            base = s * RPT + j * CHUNK
            init_handles.append(pltpu.async_copy(
                bufs[0], agg_sh.at[pl.ds(base, CHUNK)], gsems[0]))
            init_handles.append(pltpu.async_copy(
                zer16, deg_sh.at[pl.ds(base, CHUNK)], gsems[1]))
        for h in init_handles:
            h.wait()

        plsc.subcore_barrier()

        def run_loop(x_hbm, deg_par):
            rows0, rows1 = bufs[0], bufs[1]
            sem0, sem1 = gsems[0], gsems[1]
            pltpu.async_copy(x_hbm.at[srcb.at[0]], rows0, sem0)

            def pair(p, carry):
                t0 = 2 * p
                g1 = pltpu.async_copy(x_hbm.at[srcb.at[t0 + 1]], rows1, sem1)
                pltpu.make_async_copy(x_hbm.at[srcb.at[t0]], rows0,
                                      sem0).wait()
                pltpu.sync_copy(rows0, agg_sh.at[dstb.at[t0]], add=True)
                if deg_par == 0:
                    pltpu.sync_copy(ones, deg_sh.at[dstb.at[t0]], add=True)
                pltpu.async_copy(x_hbm.at[srcb.at[t0 + 2]], rows0, sem0)
                g1.wait()
                pltpu.sync_copy(rows1, agg_sh.at[dstb.at[t0 + 1]], add=True)
                if deg_par == 1:
                    pltpu.sync_copy(ones, deg_sh.at[dstb.at[t0 + 1]],
                                    add=True)
                return carry
            lax.fori_loop(0, NCHUNK // 2, pair, 0)
            pltpu.make_async_copy(x_hbm.at[srcb.at[NCHUNK]], rows0,
                                  sem0).wait()

        @pl.when(c == 0)
        def _():
            run_loop(x0_hbm, 0)

        @pl.when(c == 1)
        def _():
            run_loop(x1_hbm, 1)

        plsc.subcore_barrier()

        wb0 = pltpu.async_copy(agg_sh.at[pl.ds(s * RPT, RPT)],
                               agg_out.at[c, pl.ds(s * RPT, RPT)], gsems[0])
        wb1 = pltpu.async_copy(deg_sh.at[pl.ds(s * RPT, RPT)],
                               deg_out.at[c, pl.ds(s * RPT, RPT)], gsems[1])
        wb0.wait()
        wb1.wait()

    return body(x0, x1, eidx4)


BN = 2000


def _tc_split(x):
    def body(x_ref, o0_ref, o1_ref):
        o0_ref[...] = x_ref[:, 0:DH]
        o1_ref[...] = x_ref[:, DH:D]

    return pl.pallas_call(
        body,
        grid=(N // BN,),
        in_specs=[pl.BlockSpec((BN, D), lambda i: (i, 0))],
        out_specs=[pl.BlockSpec((BN, DH), lambda i: (i, 0)),
                   pl.BlockSpec((BN, DH), lambda i: (i, 0))],
        out_shape=[jax.ShapeDtypeStruct((N, DH), jnp.float32),
                   jax.ShapeDtypeStruct((N, DH), jnp.float32)],
    )(x)


def _tc_combine(aggsum, degbuf, x, wl_t, wr_t, b_row):
    def body(agg_ref, deg_ref, x_ref, wl_ref, wr_ref, b_ref, o_ref):
        d = deg_ref[0, :, 0:1] + deg_ref[1, :, 0:1]
        inv = 1.0 / jnp.maximum(d, 1.0)
        a0 = agg_ref[0] * inv
        a1 = agg_ref[1] * inv
        out = jnp.dot(a0, wl_ref[0:DH, :], preferred_element_type=jnp.float32)
        out = out + jnp.dot(a1, wl_ref[DH:D, :],
                            preferred_element_type=jnp.float32)
        out = out + jnp.dot(x_ref[...], wr_ref[...],
                            preferred_element_type=jnp.float32)
        out = out + b_ref[...]
        o_ref[...] = jnp.maximum(out, 0.0)

    return pl.pallas_call(
        body,
        grid=(N // BN,),
        in_specs=[
            pl.BlockSpec((NC, BN, DH), lambda i: (0, i, 0)),
            pl.BlockSpec((NC, BN, 16), lambda i: (0, i, 0)),
            pl.BlockSpec((BN, D), lambda i: (i, 0)),
            pl.BlockSpec((D, D), lambda i: (0, 0)),
            pl.BlockSpec((D, D), lambda i: (0, 0)),
            pl.BlockSpec((1, D), lambda i: (0, 0)),
        ],
        out_specs=pl.BlockSpec((BN, D), lambda i: (i, 0)),
        out_shape=jax.ShapeDtypeStruct((N, D), jnp.float32),
    )(aggsum, degbuf, x, wl_t, wr_t, b_row)


@jax.jit
def kernel(x, edge_index, W_l, b_l, W_r):
    x0, x1 = _tc_split(x)
    eidx4 = edge_index.reshape(2, NS, NCHUNK, CHUNK)
    aggsum, degbuf = _sc_aggregate(x0, x1, eidx4)
    return _tc_combine(aggsum, degbuf, x, W_l.T, W_r.T, b_l.reshape(1, D))
